# baseline (speedup 1.0000x reference)
import functools
import math

import jax
import jax.numpy as jnp
from jax.experimental import pallas as pl
from jax.experimental.pallas import tpu as pltpu

_VOCAB = 100000
_PAD = 0
_SMOOTH = 0.1
_CONF = 1.0 - _SMOOTH
_FILL = _SMOOTH / (_VOCAB - 2)
_C = _CONF * math.log(_CONF) + _SMOOTH * math.log(_FILL)

_BLOCK_COLS = 4096


def _rowsum_kernel(y_ref, acc_ref, z_ref, *, block_cols, vocab, n_blocks):
    k = pl.program_id(0)

    @pl.when(k == 0)
    def _():
        z_ref[...] = y_ref[:, 0:1]

    x = y_ref[...]

    @pl.when(k < n_blocks - 1)
    def _():
        p = jnp.sum(x, axis=1, keepdims=True)

        @pl.when(k == 0)
        def _():
            acc_ref[...] = p

        @pl.when(k != 0)
        def _():
            acc_ref[...] += p

    @pl.when(k == n_blocks - 1)
    def _():
        col = k * block_cols + jax.lax.broadcasted_iota(
            jnp.int32, (1, block_cols), 1)
        xm = jnp.where(col < vocab, x, 0.0)
        acc_ref[...] += jnp.sum(xm, axis=1, keepdims=True)


def kernel(yhat, target):
    n, vocab = yhat.shape
    t = target.astype(jnp.int32)
    n_blocks = pl.cdiv(vocab, _BLOCK_COLS)
    rowsum, z = pl.pallas_call(
        functools.partial(_rowsum_kernel, block_cols=_BLOCK_COLS,
                          vocab=vocab, n_blocks=n_blocks),
        grid=(n_blocks,),
        in_specs=[
            pl.BlockSpec((n, _BLOCK_COLS), lambda k: (0, k)),
        ],
        out_specs=[
            pl.BlockSpec((n, 1), lambda k: (0, 0)),
            pl.BlockSpec((n, 1), lambda k: (0, 0)),
        ],
        out_shape=[
            jax.ShapeDtypeStruct((n, 1), jnp.float32),
            jax.ShapeDtypeStruct((n, 1), jnp.float32),
        ],
    )(yhat)

    g = jnp.take_along_axis(yhat, t[:, None], axis=1)
    valid = (t != _PAD).astype(jnp.float32)[:, None]
    per_row = _C - (_CONF - _FILL) * g - _FILL * (rowsum - z)
    return jnp.sum(per_row * valid)
